# baseline (speedup 1.0000x reference)
import functools

import jax
import jax.numpy as jnp
from jax import lax
from jax.experimental import pallas as pl
from jax.experimental.pallas import tpu as pltpu
from jax.experimental.pallas import tpu_sc as plsc

_D_MODEL = 384
_VOCAB = 300
_VPAD = 384
_TAIL = _VOCAB - 256

_NC = 2
_NS = 16
_NW = _NC * _NS
_CHUNK = 128

_TC_BLK = 512


def _table_body(embed_ref, w1_ref, b1_ref, w2_ref, b2_ref, out_ref):
    h = lax.dot(embed_ref[...], w1_ref[...],
                precision=lax.Precision.HIGHEST,
                preferred_element_type=jnp.float32) + b1_ref[...]
    out_ref[...] = lax.dot(h, w2_ref[...],
                           precision=lax.Precision.HIGHEST,
                           preferred_element_type=jnp.float32) + b2_ref[...]


def _make_table(embed, W1, b1, W2, b2):
    W2p = jnp.pad(W2, ((0, 0), (0, _VPAD - _VOCAB)))
    b2p = jnp.pad(b2, (0, _VPAD - _VOCAB))
    return pl.pallas_call(
        _table_body,
        out_shape=jax.ShapeDtypeStruct((_VOCAB, _VPAD), jnp.float32),
    )(embed, W1, b1.reshape(1, _D_MODEL), W2p, b2p.reshape(1, _VPAD))


def _tail_fill(rows_ref, tail_ref):
    def body(r, carry):
        for src, dst in ((256, 0), (272, 16), (284, 28)):
            tail_ref[r, pl.ds(dst, 16)] = rows_ref[r, pl.ds(src, 16)]
        return carry

    lax.fori_loop(0, _CHUNK, body, 0)


def _make_gather(n_total, n_active):
    per_w = n_active // _NW
    n_chunks = per_w // _CHUNK
    mesh = plsc.VectorSubcoreMesh(core_axis_name="c", subcore_axis_name="s")

    @functools.partial(
        pl.kernel, mesh=mesh,
        out_type=jax.ShapeDtypeStruct((n_total, _VOCAB), jnp.float32),
        scratch_types=[
            pltpu.VMEM((n_chunks, _CHUNK), jnp.int32),
            pltpu.VMEM((_CHUNK, _VPAD), jnp.float32),
            pltpu.VMEM((_CHUNK, _VPAD), jnp.float32),
            pltpu.VMEM((_CHUNK, _TAIL), jnp.float32),
            pltpu.SemaphoreType.DMA,
            pltpu.SemaphoreType.DMA,
            pltpu.SemaphoreType.DMA,
            pltpu.SemaphoreType.DMA,
            pltpu.SemaphoreType.DMA,
        ],
    )
    def gather(table_hbm, idx_hbm, out_hbm, idx_all, rows0, rows1, tail_v,
               sg0, sg1, sm0, sm1, st):
        wid = lax.axis_index("s") * _NC + lax.axis_index("c")
        base = wid * per_w
        rows = (rows0, rows1)
        semg = (sg0, sg1)
        semm = (sm0, sm1)
        pltpu.sync_copy(idx_hbm.at[pl.ds(wid * n_chunks, n_chunks)], idx_all)

        gath = [None, None]
        wmain = [None, None]
        wtail = None
        for c in range(n_chunks + 1):
            if c < n_chunks:
                b = c & 1
                if wmain[b] is not None:
                    wmain[b].wait()
                gath[b] = pltpu.async_copy(
                    table_hbm.at[idx_all.at[c]], rows[b], semg[b])
            if c >= 1:
                p = (c - 1) & 1
                off = base + (c - 1) * _CHUNK
                gath[p].wait()
                if wtail is not None:
                    wtail.wait()
                _tail_fill(rows[p], tail_v)
                wmain[p] = pltpu.async_copy(
                    rows[p].at[:, pl.ds(0, 256)],
                    out_hbm.at[pl.ds(off, _CHUNK), pl.ds(0, 256)], semm[p])
                wtail = pltpu.async_copy(
                    tail_v, out_hbm.at[pl.ds(off, _CHUNK), pl.ds(256, _TAIL)],
                    st)
        for b in range(2):
            wmain[b].wait()
        wtail.wait()

    return gather


def _onehot_body(ids_ref, t_ref, out_ref):
    ids = ids_ref[0, 0, :]
    iota = lax.broadcasted_iota(jnp.int32, (_TC_BLK, _VPAD), 1)
    oh = (ids[:, None] == iota).astype(jnp.bfloat16)
    out_ref[...] = lax.dot(oh, t_ref[...],
                           preferred_element_type=jnp.float32)


def _onehot_lookup(table_bf, ids_tc):
    n = ids_tc.shape[0]
    nblk = n // _TC_BLK
    ids3 = ids_tc.reshape(nblk, 1, _TC_BLK)
    return pl.pallas_call(
        _onehot_body,
        grid=(nblk,),
        in_specs=[
            pl.BlockSpec((1, 1, _TC_BLK), lambda i: (i, 0, 0)),
            pl.BlockSpec((_VPAD, _VOCAB), lambda i: (0, 0)),
        ],
        out_specs=pl.BlockSpec((_TC_BLK, _VOCAB), lambda i: (i, 0)),
        out_shape=jax.ShapeDtypeStruct((n, _VOCAB), jnp.float32),
    )(ids3, table_bf)


def kernel(byte_input, embed, W1, b1, W2, b2):
    batch, seq = byte_input.shape
    half = seq // 2
    n_sc = batch * half
    table = _make_table(embed, W1, b1, W2, b2)
    ids = byte_input.astype(jnp.int32)
    ids_sc = ids[:, :half].reshape(n_sc // _CHUNK, _CHUNK)
    ids_tc = ids[:, half:].reshape(-1)
    out_sc = _make_gather(n_sc, n_sc)(table, ids_sc)
    table_bf = jnp.pad(table[:, :_VOCAB].astype(jnp.bfloat16),
                       ((0, _VPAD - _VOCAB), (0, 0)))
    out_tc = _onehot_lookup(table_bf, ids_tc)
    return jnp.concatenate(
        [out_sc.reshape(batch, half, _VOCAB),
         out_tc.reshape(batch, half, _VOCAB)], axis=1)


# speedup vs baseline: 1.1801x; 1.1801x over previous
import functools

import jax
import jax.numpy as jnp
from jax import lax
from jax.experimental import pallas as pl
from jax.experimental.pallas import tpu as pltpu
from jax.experimental.pallas import tpu_sc as plsc

_D_MODEL = 384
_VOCAB = 300
_VPAD = 384
_TAIL = _VOCAB - 256

_NC = 2
_NS = 16
_NW = _NC * _NS
_CHUNK = 128

_TC_BLK = 1024


def _table_body(embed_ref, w1_ref, b1_ref, w2_ref, b2_ref, out_ref):
    h = lax.dot(embed_ref[...], w1_ref[...],
                precision=lax.Precision.HIGHEST,
                preferred_element_type=jnp.float32) + b1_ref[...]
    out_ref[...] = lax.dot(h, w2_ref[...],
                           precision=lax.Precision.HIGHEST,
                           preferred_element_type=jnp.float32) + b2_ref[...]


def _make_table(embed, W1, b1, W2, b2):
    W2p = jnp.pad(W2, ((0, 0), (0, _VPAD - _VOCAB)))
    b2p = jnp.pad(b2, (0, _VPAD - _VOCAB))
    return pl.pallas_call(
        _table_body,
        out_shape=jax.ShapeDtypeStruct((_VOCAB, _VPAD), jnp.float32),
    )(embed, W1, b1.reshape(1, _D_MODEL), W2p, b2p.reshape(1, _VPAD))


def _tail_fill(rows_ref, tail_ref):
    def body(r, carry):
        for src, dst in ((256, 0), (272, 16), (284, 28)):
            tail_ref[r, pl.ds(dst, 16)] = rows_ref[r, pl.ds(src, 16)]
        return carry

    lax.fori_loop(0, _CHUNK, body, 0)


def _make_gather(n_total, n_active):
    per_w = n_active // _NW
    n_chunks = per_w // _CHUNK
    mesh = plsc.VectorSubcoreMesh(core_axis_name="c", subcore_axis_name="s")

    @functools.partial(
        pl.kernel, mesh=mesh,
        out_type=jax.ShapeDtypeStruct((n_total, _VOCAB), jnp.float32),
        scratch_types=[
            pltpu.VMEM((n_chunks, _CHUNK), jnp.int32),
            pltpu.VMEM((_CHUNK, _VPAD), jnp.float32),
            pltpu.VMEM((_CHUNK, _VPAD), jnp.float32),
            pltpu.VMEM((_CHUNK, _TAIL), jnp.float32),
            pltpu.SemaphoreType.DMA,
            pltpu.SemaphoreType.DMA,
            pltpu.SemaphoreType.DMA,
            pltpu.SemaphoreType.DMA,
            pltpu.SemaphoreType.DMA,
        ],
    )
    def gather(table_hbm, idx_hbm, out_hbm, idx_all, rows0, rows1, tail_v,
               sg0, sg1, sm0, sm1, st):
        wid = lax.axis_index("s") * _NC + lax.axis_index("c")
        base = wid * per_w
        rows = (rows0, rows1)
        semg = (sg0, sg1)
        semm = (sm0, sm1)
        pltpu.sync_copy(idx_hbm.at[pl.ds(wid * n_chunks, n_chunks)], idx_all)

        gath = [None, None]
        wmain = [None, None]
        wtail = None
        for c in range(n_chunks + 1):
            if c < n_chunks:
                b = c & 1
                if wmain[b] is not None:
                    wmain[b].wait()
                gath[b] = pltpu.async_copy(
                    table_hbm.at[idx_all.at[c]], rows[b], semg[b])
            if c >= 1:
                p = (c - 1) & 1
                off = base + (c - 1) * _CHUNK
                gath[p].wait()
                if wtail is not None:
                    wtail.wait()
                _tail_fill(rows[p], tail_v)
                wmain[p] = pltpu.async_copy(
                    rows[p].at[:, pl.ds(0, 256)],
                    out_hbm.at[pl.ds(off, _CHUNK), pl.ds(0, 256)], semm[p])
                wtail = pltpu.async_copy(
                    tail_v, out_hbm.at[pl.ds(off, _CHUNK), pl.ds(256, _TAIL)],
                    st)
        for b in range(2):
            wmain[b].wait()
        wtail.wait()

    return gather


def _onehot_body(ids_ref, t_ref, _buf_ref, out_ref):
    ids = ids_ref[0, 0, :]
    iota = lax.broadcasted_iota(jnp.int32, (_TC_BLK, _VPAD), 1)
    oh = (ids[:, None] == iota).astype(jnp.bfloat16)
    out_ref[...] = lax.dot(oh, t_ref[...],
                           preferred_element_type=jnp.float32)


def _onehot_fill(table_bf, ids_tc, buf, n_sc):
    n_total = buf.shape[0]
    nblk = (n_total - n_sc) // _TC_BLK
    blk0 = n_sc // _TC_BLK
    ids3 = ids_tc.reshape(nblk, 1, _TC_BLK)
    return pl.pallas_call(
        _onehot_body,
        grid=(nblk,),
        in_specs=[
            pl.BlockSpec((1, 1, _TC_BLK), lambda i: (i, 0, 0)),
            pl.BlockSpec((_VPAD, _VOCAB), lambda i: (0, 0)),
            pl.BlockSpec(memory_space=pl.MemorySpace.ANY),
        ],
        out_specs=pl.BlockSpec((_TC_BLK, _VOCAB), lambda i: (i + blk0, 0)),
        out_shape=jax.ShapeDtypeStruct((n_total, _VOCAB), jnp.float32),
        input_output_aliases={2: 0},
    )(ids3, table_bf, buf)


def kernel(byte_input, embed, W1, b1, W2, b2):
    batch, seq = byte_input.shape
    n_tokens = batch * seq
    n_sc = n_tokens // 2
    table = _make_table(embed, W1, b1, W2, b2)
    ids = byte_input.reshape(-1).astype(jnp.int32)
    ids_sc = ids[:n_sc].reshape(n_sc // _CHUNK, _CHUNK)
    buf = _make_gather(n_tokens, n_sc)(table, ids_sc)
    table_bf = jnp.pad(table[:, :_VOCAB].astype(jnp.bfloat16),
                       ((0, _VPAD - _VOCAB), (0, 0)))
    out = _onehot_fill(table_bf, ids[n_sc:], buf, n_sc)
    return out.reshape(batch, seq, _VOCAB)


# speedup vs baseline: 1.2471x; 1.0568x over previous
import functools

import jax
import jax.numpy as jnp
from jax import lax
from jax.experimental import pallas as pl
from jax.experimental.pallas import tpu as pltpu
from jax.experimental.pallas import tpu_sc as plsc

_D_MODEL = 384
_VOCAB = 300
_VPAD = 384
_TAIL = _VOCAB - 256

_NC = 2
_NS = 16
_NW = _NC * _NS
_CHUNK = 128

_TC_BLK = 1024


def _table_body(embed_ref, w1_ref, b1_ref, w2_ref, b2_ref, out_ref):
    h = lax.dot(embed_ref[...], w1_ref[...],
                precision=lax.Precision.HIGHEST,
                preferred_element_type=jnp.float32) + b1_ref[...]
    out_ref[...] = lax.dot(h, w2_ref[...],
                           precision=lax.Precision.HIGHEST,
                           preferred_element_type=jnp.float32) + b2_ref[...]


def _make_table(embed, W1, b1, W2, b2):
    W2p = jnp.pad(W2, ((0, 0), (0, _VPAD - _VOCAB)))
    b2p = jnp.pad(b2, (0, _VPAD - _VOCAB))
    return pl.pallas_call(
        _table_body,
        out_shape=jax.ShapeDtypeStruct((_VOCAB, _VPAD), jnp.float32),
    )(embed, W1, b1.reshape(1, _D_MODEL), W2p, b2p.reshape(1, _VPAD))


def _tail_fill(rows_ref, tail_ref):
    def body(r, carry):
        for src, dst in ((256, 0), (272, 16), (284, 28)):
            tail_ref[r, pl.ds(dst, 16)] = rows_ref[r, pl.ds(src, 16)]
        return carry

    lax.fori_loop(0, _CHUNK, body, 0)


def _make_gather(n_total, n_active):
    per_w = n_active // _NW
    n_chunks = per_w // _CHUNK
    mesh = plsc.VectorSubcoreMesh(core_axis_name="c", subcore_axis_name="s")

    @functools.partial(
        pl.kernel, mesh=mesh,
        out_type=jax.ShapeDtypeStruct((n_total, _VOCAB), jnp.float32),
        scratch_types=[
            pltpu.VMEM((per_w,), jnp.int32),
            pltpu.VMEM((_CHUNK, _VPAD), jnp.float32),
            pltpu.VMEM((_CHUNK, _VPAD), jnp.float32),
            pltpu.VMEM((_CHUNK, _TAIL), jnp.float32),
            pltpu.SemaphoreType.DMA,
            pltpu.SemaphoreType.DMA,
            pltpu.SemaphoreType.DMA,
            pltpu.SemaphoreType.DMA,
            pltpu.SemaphoreType.DMA,
        ],
    )
    def gather(table_hbm, idx_hbm, out_hbm, idx_all, rows0, rows1, tail_v,
               sg0, sg1, sm0, sm1, st):
        wid = lax.axis_index("s") * _NC + lax.axis_index("c")
        base = wid * per_w
        rows = (rows0, rows1)
        semg = (sg0, sg1)
        semm = (sm0, sm1)
        pltpu.sync_copy(idx_hbm.at[pl.ds(base, per_w)], idx_all)

        gath = [None, None]
        wmain = [None, None]
        wtail = None
        for c in range(n_chunks + 1):
            if c < n_chunks:
                b = c & 1
                if wmain[b] is not None:
                    wmain[b].wait()
                gath[b] = pltpu.async_copy(
                    table_hbm.at[idx_all.at[pl.ds(c * _CHUNK, _CHUNK)]],
                    rows[b], semg[b])
            if c >= 1:
                p = (c - 1) & 1
                off = base + (c - 1) * _CHUNK
                gath[p].wait()
                if wtail is not None:
                    wtail.wait()
                _tail_fill(rows[p], tail_v)
                wmain[p] = pltpu.async_copy(
                    rows[p].at[:, pl.ds(0, 256)],
                    out_hbm.at[pl.ds(off, _CHUNK), pl.ds(0, 256)], semm[p])
                wtail = pltpu.async_copy(
                    tail_v, out_hbm.at[pl.ds(off, _CHUNK), pl.ds(256, _TAIL)],
                    st)
        for b in range(2):
            wmain[b].wait()
        wtail.wait()

    return gather


def _onehot_body(ids_ref, t_ref, _buf_ref, out_ref):
    ids = ids_ref[0, 0, :]
    iota = lax.broadcasted_iota(jnp.int32, (_TC_BLK, _VPAD), 1)
    oh = (ids[:, None] == iota).astype(jnp.bfloat16)
    out_ref[...] = lax.dot(oh, t_ref[...],
                           preferred_element_type=jnp.float32)


def _onehot_fill(table_bf, ids_tc, buf, n_sc):
    n_total = buf.shape[0]
    nblk = (n_total - n_sc) // _TC_BLK
    blk0 = n_sc // _TC_BLK
    ids3 = ids_tc.reshape(nblk, 1, _TC_BLK)
    return pl.pallas_call(
        _onehot_body,
        grid=(nblk,),
        in_specs=[
            pl.BlockSpec((1, 1, _TC_BLK), lambda i: (i, 0, 0)),
            pl.BlockSpec((_VPAD, _VOCAB), lambda i: (0, 0)),
            pl.BlockSpec(memory_space=pl.MemorySpace.ANY),
        ],
        out_specs=pl.BlockSpec((_TC_BLK, _VOCAB), lambda i: (i + blk0, 0)),
        out_shape=jax.ShapeDtypeStruct((n_total, _VOCAB), jnp.float32),
        input_output_aliases={2: 0},
    )(ids3, table_bf, buf)


def kernel(byte_input, embed, W1, b1, W2, b2):
    batch, seq = byte_input.shape
    n_tokens = batch * seq
    n_sc = (3 * n_tokens) // 8
    table = _make_table(embed, W1, b1, W2, b2)
    ids = byte_input.reshape(-1).astype(jnp.int32)
    ids_sc = ids[:n_sc]
    buf = _make_gather(n_tokens, n_sc)(table, ids_sc)
    table_bf = jnp.pad(table[:, :_VOCAB].astype(jnp.bfloat16),
                       ((0, _VPAD - _VOCAB), (0, 0)))
    out = _onehot_fill(table_bf, ids[n_sc:], buf, n_sc)
    return out.reshape(batch, seq, _VOCAB)


# speedup vs baseline: 1.3524x; 1.0845x over previous
import functools

import jax
import jax.numpy as jnp
from jax import lax
from jax.experimental import pallas as pl
from jax.experimental.pallas import tpu as pltpu
from jax.experimental.pallas import tpu_sc as plsc

_D_MODEL = 384
_VOCAB = 300
_VPAD = 384
_TAIL = _VOCAB - 256

_NC = 2
_NS = 16
_NW = _NC * _NS
_CHUNK = 128

_TC_BLK = 2048


def _table_body(embed_ref, w1_ref, b1_ref, w2_ref, b2_ref, out_ref):
    h = lax.dot(embed_ref[...], w1_ref[...],
                precision=lax.Precision.HIGHEST,
                preferred_element_type=jnp.float32) + b1_ref[...]
    out_ref[...] = lax.dot(h, w2_ref[...],
                           precision=lax.Precision.HIGHEST,
                           preferred_element_type=jnp.float32) + b2_ref[...]


def _make_table(embed, W1, b1, W2, b2):
    W2p = jnp.pad(W2, ((0, 0), (0, _VPAD - _VOCAB)))
    b2p = jnp.pad(b2, (0, _VPAD - _VOCAB))
    return pl.pallas_call(
        _table_body,
        out_shape=jax.ShapeDtypeStruct((_VOCAB, _VPAD), jnp.float32),
    )(embed, W1, b1.reshape(1, _D_MODEL), W2p, b2p.reshape(1, _VPAD))


def _tail_fill(rows_ref, tail_ref):
    def body(r, carry):
        for src, dst in ((256, 0), (272, 16), (284, 28)):
            tail_ref[r, pl.ds(dst, 16)] = rows_ref[r, pl.ds(src, 16)]
        return carry

    lax.fori_loop(0, _CHUNK, body, 0)


def _make_gather(n_total, n_active):
    per_w = n_active // _NW
    n_chunks = per_w // _CHUNK
    mesh = plsc.VectorSubcoreMesh(core_axis_name="c", subcore_axis_name="s")

    @functools.partial(
        pl.kernel, mesh=mesh,
        out_type=jax.ShapeDtypeStruct((n_total, _VOCAB), jnp.float32),
        scratch_types=[
            pltpu.VMEM((per_w,), jnp.int32),
            pltpu.VMEM((_CHUNK, _VPAD), jnp.float32),
            pltpu.VMEM((_CHUNK, _VPAD), jnp.float32),
            pltpu.VMEM((_CHUNK, _TAIL), jnp.float32),
            pltpu.SemaphoreType.DMA,
            pltpu.SemaphoreType.DMA,
            pltpu.SemaphoreType.DMA,
            pltpu.SemaphoreType.DMA,
            pltpu.SemaphoreType.DMA,
        ],
    )
    def gather(table_hbm, idx_hbm, out_hbm, idx_all, rows0, rows1, tail_v,
               sg0, sg1, sm0, sm1, st):
        wid = lax.axis_index("s") * _NC + lax.axis_index("c")
        base = wid * per_w
        rows = (rows0, rows1)
        semg = (sg0, sg1)
        semm = (sm0, sm1)
        pltpu.sync_copy(idx_hbm.at[pl.ds(base, per_w)], idx_all)

        gath = [None, None]
        wmain = [None, None]
        wtail = None
        for c in range(n_chunks + 1):
            if c < n_chunks:
                b = c & 1
                if wmain[b] is not None:
                    wmain[b].wait()
                gath[b] = pltpu.async_copy(
                    table_hbm.at[idx_all.at[pl.ds(c * _CHUNK, _CHUNK)]],
                    rows[b], semg[b])
            if c >= 1:
                p = (c - 1) & 1
                off = base + (c - 1) * _CHUNK
                gath[p].wait()
                if wtail is not None:
                    wtail.wait()
                _tail_fill(rows[p], tail_v)
                wmain[p] = pltpu.async_copy(
                    rows[p].at[:, pl.ds(0, 256)],
                    out_hbm.at[pl.ds(off, _CHUNK), pl.ds(0, 256)], semm[p])
                wtail = pltpu.async_copy(
                    tail_v, out_hbm.at[pl.ds(off, _CHUNK), pl.ds(256, _TAIL)],
                    st)
        for b in range(2):
            wmain[b].wait()
        wtail.wait()

    return gather


def _onehot_body(ids_ref, t_ref, _buf_ref, out_ref):
    ids = ids_ref[0, 0, :]
    iota = lax.broadcasted_iota(jnp.int32, (_TC_BLK, _VPAD), 1)
    oh = (ids[:, None] == iota).astype(jnp.bfloat16)
    out_ref[...] = lax.dot(oh, t_ref[...],
                           preferred_element_type=jnp.float32)


def _onehot_fill(table_bf, ids_tc, buf, n_sc):
    n_total = buf.shape[0]
    nblk = (n_total - n_sc) // _TC_BLK
    blk0 = n_sc // _TC_BLK
    ids3 = ids_tc.reshape(nblk, 1, _TC_BLK)
    return pl.pallas_call(
        _onehot_body,
        grid=(nblk,),
        in_specs=[
            pl.BlockSpec((1, 1, _TC_BLK), lambda i: (i, 0, 0)),
            pl.BlockSpec((_VPAD, _VOCAB), lambda i: (0, 0)),
            pl.BlockSpec(memory_space=pl.MemorySpace.ANY),
        ],
        out_specs=pl.BlockSpec((_TC_BLK, _VOCAB), lambda i: (i + blk0, 0)),
        out_shape=jax.ShapeDtypeStruct((n_total, _VOCAB), jnp.float32),
        input_output_aliases={2: 0},
    )(ids3, table_bf, buf)


def kernel(byte_input, embed, W1, b1, W2, b2):
    batch, seq = byte_input.shape
    n_tokens = batch * seq
    n_sc = n_tokens // 4
    table = _make_table(embed, W1, b1, W2, b2)
    ids = byte_input.reshape(-1).astype(jnp.int32)
    ids_sc = ids[:n_sc]
    buf = _make_gather(n_tokens, n_sc)(table, ids_sc)
    table_bf = jnp.pad(table[:, :_VOCAB].astype(jnp.bfloat16),
                       ((0, _VPAD - _VOCAB), (0, 0)))
    out = _onehot_fill(table_bf, ids[n_sc:], buf, n_sc)
    return out.reshape(batch, seq, _VOCAB)


# speedup vs baseline: 1.3767x; 1.0180x over previous
import functools

import jax
import jax.numpy as jnp
from jax import lax
from jax.experimental import pallas as pl
from jax.experimental.pallas import tpu as pltpu
from jax.experimental.pallas import tpu_sc as plsc

_D_MODEL = 384
_VOCAB = 300
_VPAD = 384
_TAIL = _VOCAB - 256

_NC = 2
_NS = 16
_NW = _NC * _NS
_CHUNK = 128

_TC_BLK = 4096


def _table_body(embed_ref, w1_ref, b1_ref, w2_ref, b2_ref, out_ref):
    h = lax.dot(embed_ref[...], w1_ref[...],
                precision=lax.Precision.HIGHEST,
                preferred_element_type=jnp.float32) + b1_ref[...]
    out_ref[...] = lax.dot(h, w2_ref[...],
                           precision=lax.Precision.HIGHEST,
                           preferred_element_type=jnp.float32) + b2_ref[...]


def _make_table(embed, W1, b1, W2, b2):
    W2p = jnp.pad(W2, ((0, 0), (0, _VPAD - _VOCAB)))
    b2p = jnp.pad(b2, (0, _VPAD - _VOCAB))
    return pl.pallas_call(
        _table_body,
        out_shape=jax.ShapeDtypeStruct((_VOCAB, _VPAD), jnp.float32),
    )(embed, W1, b1.reshape(1, _D_MODEL), W2p, b2p.reshape(1, _VPAD))


def _tail_fill(rows_ref, tail_ref):
    def body(r, carry):
        for src, dst in ((256, 0), (272, 16), (284, 28)):
            tail_ref[r, pl.ds(dst, 16)] = rows_ref[r, pl.ds(src, 16)]
        return carry

    lax.fori_loop(0, _CHUNK, body, 0)


def _make_gather(n_total, n_active):
    per_w = n_active // _NW
    n_chunks = per_w // _CHUNK
    mesh = plsc.VectorSubcoreMesh(core_axis_name="c", subcore_axis_name="s")

    @functools.partial(
        pl.kernel, mesh=mesh,
        out_type=jax.ShapeDtypeStruct((n_total, _VOCAB), jnp.float32),
        scratch_types=[
            pltpu.VMEM((per_w,), jnp.int32),
            pltpu.VMEM((_CHUNK, _VPAD), jnp.float32),
            pltpu.VMEM((_CHUNK, _VPAD), jnp.float32),
            pltpu.VMEM((_CHUNK, _TAIL), jnp.float32),
            pltpu.SemaphoreType.DMA,
            pltpu.SemaphoreType.DMA,
            pltpu.SemaphoreType.DMA,
            pltpu.SemaphoreType.DMA,
            pltpu.SemaphoreType.DMA,
        ],
    )
    def gather(table_hbm, idx_hbm, out_hbm, idx_all, rows0, rows1, tail_v,
               sg0, sg1, sm0, sm1, st):
        wid = lax.axis_index("s") * _NC + lax.axis_index("c")
        base = wid * per_w
        rows = (rows0, rows1)
        semg = (sg0, sg1)
        semm = (sm0, sm1)
        pltpu.sync_copy(idx_hbm.at[pl.ds(base, per_w)], idx_all)

        gath = [None, None]
        wmain = [None, None]
        wtail = None
        for c in range(n_chunks + 1):
            if c < n_chunks:
                b = c & 1
                if wmain[b] is not None:
                    wmain[b].wait()
                gath[b] = pltpu.async_copy(
                    table_hbm.at[idx_all.at[pl.ds(c * _CHUNK, _CHUNK)]],
                    rows[b], semg[b])
            if c >= 1:
                p = (c - 1) & 1
                off = base + (c - 1) * _CHUNK
                gath[p].wait()
                if wtail is not None:
                    wtail.wait()
                _tail_fill(rows[p], tail_v)
                wmain[p] = pltpu.async_copy(
                    rows[p].at[:, pl.ds(0, 256)],
                    out_hbm.at[pl.ds(off, _CHUNK), pl.ds(0, 256)], semm[p])
                wtail = pltpu.async_copy(
                    tail_v, out_hbm.at[pl.ds(off, _CHUNK), pl.ds(256, _TAIL)],
                    st)
        for b in range(2):
            wmain[b].wait()
        wtail.wait()

    return gather


def _onehot_body(ids_ref, t_ref, _buf_ref, out_ref):
    ids = ids_ref[0, 0, :]
    iota = lax.broadcasted_iota(jnp.int32, (_TC_BLK, _VPAD), 1)
    oh = (ids[:, None] == iota).astype(jnp.bfloat16)
    out_ref[...] = lax.dot(oh, t_ref[...],
                           preferred_element_type=jnp.float32)


def _onehot_fill(table_bf, ids_tc, buf, n_sc):
    n_total = buf.shape[0]
    nblk = (n_total - n_sc) // _TC_BLK
    blk0 = n_sc // _TC_BLK
    ids3 = ids_tc.reshape(nblk, 1, _TC_BLK)
    return pl.pallas_call(
        _onehot_body,
        grid=(nblk,),
        in_specs=[
            pl.BlockSpec((1, 1, _TC_BLK), lambda i: (i, 0, 0)),
            pl.BlockSpec((_VPAD, _VOCAB), lambda i: (0, 0)),
            pl.BlockSpec(memory_space=pl.MemorySpace.ANY),
        ],
        out_specs=pl.BlockSpec((_TC_BLK, _VOCAB), lambda i: (i + blk0, 0)),
        out_shape=jax.ShapeDtypeStruct((n_total, _VOCAB), jnp.float32),
        input_output_aliases={2: 0},
    )(ids3, table_bf, buf)


def kernel(byte_input, embed, W1, b1, W2, b2):
    batch, seq = byte_input.shape
    n_tokens = batch * seq
    n_sc = n_tokens // 4
    table = _make_table(embed, W1, b1, W2, b2)
    ids = byte_input.reshape(-1).astype(jnp.int32)
    ids_sc = ids[:n_sc]
    buf = _make_gather(n_tokens, n_sc)(table, ids_sc)
    table_bf = jnp.pad(table[:, :_VOCAB].astype(jnp.bfloat16),
                       ((0, _VPAD - _VOCAB), (0, 0)))
    out = _onehot_fill(table_bf, ids[n_sc:], buf, n_sc)
    return out.reshape(batch, seq, _VOCAB)
